# Initial kernel scaffold; baseline (speedup 1.0000x reference)
import jax
import jax.numpy as jnp
from jax.experimental import pallas as pl


def kernel(x, weights):
    raise NotImplementedError("write your pallas kernel here")


# speedup vs baseline: 1.5772x; 1.5772x over previous
import functools

import jax
import jax.numpy as jnp
from jax import lax
from jax.experimental import pallas as pl
from jax.experimental.pallas import tpu as pltpu
from jax.experimental.pallas import tpu_sc as plsc

N_ROWS = 1_000_000
D = 32
B = 16384 * 26
NC, NS = 2, 16
NW = NC * NS
BPW = B // NW
CH = 1664
NCH = BPW // CH

_mesh = plsc.VectorSubcoreMesh(core_axis_name="c", subcore_axis_name="s")


@functools.partial(
    pl.kernel,
    mesh=_mesh,
    out_type=jax.ShapeDtypeStruct((B, D), jnp.float32),
    scratch_types=[
        pltpu.VMEM((BPW,), jnp.int32),
        pltpu.VMEM((2, CH, D), jnp.float32),
        pltpu.SemaphoreType.DMA,
        pltpu.SemaphoreType.DMA,
    ],
    compiler_params=pltpu.CompilerParams(use_tc_tiling_on_sc=False),
)
def _gather_kernel(table_hbm, idx_hbm, out_hbm, idx_v, rows_v, sem0, sem1):
    wid = lax.axis_index("s") * NC + lax.axis_index("c")
    base = wid * BPW
    pltpu.sync_copy(idx_hbm.at[pl.ds(base, BPW)], idx_v)

    sems = (sem0, sem1)
    copies = [None, None]
    copies[0] = pltpu.async_copy(
        table_hbm.at[idx_v.at[pl.ds(0, CH)]], rows_v.at[0], sems[0])
    for i in range(NCH):
        cur = i % 2
        nxt = (i + 1) % 2
        if i + 1 < NCH:
            copies[nxt] = pltpu.async_copy(
                table_hbm.at[idx_v.at[pl.ds((i + 1) * CH, CH)]],
                rows_v.at[nxt], sems[nxt])
        copies[cur].wait()
        pltpu.sync_copy(rows_v.at[cur], out_hbm.at[pl.ds(base + i * CH, CH)])


def kernel(x, weights):
    flat = x.reshape(-1).astype(jnp.int32)
    out = _gather_kernel(weights, flat)
    return out.reshape(x.shape + (weights.shape[1],))
